# Initial kernel scaffold; baseline (speedup 1.0000x reference)
import jax
import jax.numpy as jnp
from jax.experimental import pallas as pl


def kernel(x, edge_index, src, dst, s2d, s2dc, d2s, d2sc, W_self, W_neigh, b, node_biases, mu):
    raise NotImplementedError("write your pallas kernel here")


# speedup vs baseline: 9.6392x; 9.6392x over previous
import functools

import jax
import jax.numpy as jnp
from jax import lax
from jax.experimental import pallas as pl
from jax.experimental.pallas import tpu as pltpu
from jax.experimental.pallas import tpu_sc as plsc

N_NODES = 10000
D = 128
N_EDGES = 320000
B = 8192
P = 20

NC = 2
NS = 16
NW = NC * NS

EPW = N_EDGES // NW
ECH = 125
NCH = EPW // ECH
ICH = 16

NPAD = 10240
STRIDE = NPAD // NS
ZROWS = 32

BPW = B // NW
G = 4
NG = BPW // G

_f32 = jnp.float32


def _edge_body(x_hbm, srcL_hbm, dstL_hbm,
               msum_hbm,
               sidx_v, didx_v, rows_v, buf_v,
               msum_sh, sem):
    c = lax.axis_index("c")
    s = lax.axis_index("s")
    wid = s * NC + c

    zero16 = jnp.zeros((16,), _f32)

    def zrow(i, _):
        r = i // (D // 16)
        col = (i % (D // 16)) * 16
        buf_v[r, pl.ds(col, 16)] = zero16
        return 0

    lax.fori_loop(0, ZROWS * (D // 16), zrow, 0)

    for k in range(STRIDE // ZROWS):
        off = s * STRIDE + k * ZROWS
        pltpu.sync_copy(buf_v, msum_sh.at[pl.ds(off, ZROWS)])
    plsc.subcore_barrier()

    for blk in range(NCH // ICH):
        pltpu.sync_copy(srcL_hbm.at[wid, pl.ds(blk * ICH, ICH)], sidx_v)
        pltpu.sync_copy(dstL_hbm.at[wid, pl.ds(blk * ICH, ICH)], didx_v)

        def chunk(j, _):
            pltpu.async_copy(x_hbm.at[sidx_v.at[j]], rows_v, sem).wait()
            pltpu.sync_copy(rows_v, msum_sh.at[didx_v.at[j]], add=True)
            return 0

        lax.fori_loop(0, ICH, chunk, 0)
    plsc.subcore_barrier()

    for k in range(STRIDE // ZROWS):
        off = s * STRIDE + k * ZROWS
        pltpu.sync_copy(msum_sh.at[pl.ds(off, ZROWS)], buf_v)
        pltpu.sync_copy(buf_v, msum_hbm.at[c, pl.ds(off, ZROWS)])


_edge_call = functools.partial(
    pl.kernel,
    out_type=jax.ShapeDtypeStruct((NC, NPAD, D), _f32),
    mesh=plsc.VectorSubcoreMesh(core_axis_name="c", subcore_axis_name="s",
                                num_cores=NC, num_subcores=NS),
    compiler_params=pltpu.CompilerParams(needs_layout_passes=False),
    scratch_types=[
        pltpu.VMEM((ICH, ECH), jnp.int32),
        pltpu.VMEM((ICH, ECH), jnp.int32),
        pltpu.VMEM((ECH, D), _f32),
        pltpu.VMEM((ZROWS, D), _f32),
        pltpu.VMEM_SHARED((NPAD, D), _f32),
        pltpu.SemaphoreType.DMA,
    ],
)(_edge_body)


def _deg_body(dstF_hbm, dcnt_hbm, didx_v, deg_v):
    c = lax.axis_index("c")
    s = lax.axis_index("s")
    wid = s * NC + c

    pltpu.sync_copy(dstF_hbm.at[wid], didx_v)

    zero16 = jnp.zeros((16,), _f32)
    one16 = jnp.ones((16,), _f32)

    def zr(i, _):
        deg_v[pl.ds(i * 16, 16)] = zero16
        return 0

    lax.fori_loop(0, NPAD // 16, zr, 0)

    def chunk(i, _):
        idx = didx_v[pl.ds(i * 16, 16)]
        plsc.addupdate_scatter(deg_v, [idx], one16)
        return 0

    lax.fori_loop(0, EPW // 16, chunk, 0)

    pltpu.sync_copy(deg_v, dcnt_hbm.at[wid])


_deg_call = functools.partial(
    pl.kernel,
    out_type=jax.ShapeDtypeStruct((NW, NPAD), _f32),
    mesh=plsc.VectorSubcoreMesh(core_axis_name="c", subcore_axis_name="s",
                                num_cores=NC, num_subcores=NS),
    compiler_params=pltpu.CompilerParams(needs_layout_passes=False),
    scratch_types=[
        pltpu.VMEM((EPW,), jnp.int32),
        pltpu.VMEM((NPAD,), _f32),
    ],
)(_deg_body)


RBLK = 1024


def _h_body(x_ref, m0_ref, m1_ref, d_ref, ws_ref, wn_ref, b_ref,
            o_ref):
    i = pl.program_id(0)
    deg = jnp.sum(d_ref[...], axis=0)[:, None]
    agg = (m0_ref[...] + m1_ref[...]) / jnp.maximum(deg, 1.0)
    h = jnp.dot(x_ref[...], ws_ref[...], preferred_element_type=_f32)
    h = h + jnp.dot(agg, wn_ref[...], preferred_element_type=_f32)
    h = jnp.maximum(h + b_ref[...], 0.0)
    rows = i * RBLK + lax.broadcasted_iota(jnp.int32, (RBLK, D), 0)
    o_ref[...] = jnp.where(rows < N_NODES, h, 0.0)


_h_call = pl.pallas_call(
    _h_body,
    grid=(NPAD // RBLK,),
    in_specs=[
        pl.BlockSpec((RBLK, D), lambda i: (i, 0)),
        pl.BlockSpec((RBLK, D), lambda i: (i, 0)),
        pl.BlockSpec((RBLK, D), lambda i: (i, 0)),
        pl.BlockSpec((NW, RBLK), lambda i: (0, i)),
        pl.BlockSpec((D, D), lambda i: (0, 0)),
        pl.BlockSpec((D, D), lambda i: (0, 0)),
        pl.BlockSpec((1, D), lambda i: (0, 0)),
    ],
    out_specs=pl.BlockSpec((RBLK, D), lambda i: (i, 0)),
    out_shape=jax.ShapeDtypeStruct((NPAD, D), _f32),
)


NB_PAD = 10008
GE = 16
NBLK = BPW // GE
QROWS = GE * P // 4


def _score_body(h_hbm, src_hbm, dst_hbm, s2d_hbm, d2s_hbm, cs_hbm, cd_hbm,
                nb_hbm, mu_hbm, score_hbm,
                src_v, dst_v, s2d_v, d2s_v, cs_v, cd_v, nb_v, mu_v,
                rows_src, rows_dst, rows_s, rows_d, out_v, sem):
    c = lax.axis_index("c")
    s = lax.axis_index("s")
    wid = s * NC + c

    pltpu.sync_copy(src_hbm.at[wid], src_v)
    pltpu.sync_copy(dst_hbm.at[wid], dst_v)
    pltpu.sync_copy(s2d_hbm.at[wid], s2d_v)
    pltpu.sync_copy(d2s_hbm.at[wid], d2s_v)
    pltpu.sync_copy(cs_hbm.at[wid], cs_v)
    pltpu.sync_copy(cd_hbm.at[wid], cd_v)
    pltpu.sync_copy(nb_hbm, nb_v)
    pltpu.sync_copy(mu_hbm, mu_v)

    def remap(i, _):
        sl = pl.ds(i * 16, 16)
        v = s2d_v[sl]
        s2d_v[sl] = jnp.where(v == 0, N_NODES, v)
        w = d2s_v[sl]
        d2s_v[sl] = jnp.where(w == 0, N_NODES, w)
        return 0

    lax.fori_loop(0, (BPW * P) // 16, remap, 0)

    mu_vec = mu_v[...]

    def block(k, _):
        cps = []
        cps.append(pltpu.async_copy(h_hbm.at[src_v.at[pl.ds(k * GE, GE)]],
                                    rows_src, sem))
        cps.append(pltpu.async_copy(h_hbm.at[dst_v.at[pl.ds(k * GE, GE)]],
                                    rows_dst, sem))
        for q in range(4):
            cps.append(pltpu.async_copy(
                h_hbm.at[s2d_v.at[pl.ds(k * GE * P + q * QROWS, QROWS)]],
                rows_s.at[pl.ds(q * QROWS, QROWS)], sem))
            cps.append(pltpu.async_copy(
                h_hbm.at[d2s_v.at[pl.ds(k * GE * P + q * QROWS, QROWS)]],
                rows_d.at[pl.ds(q * QROWS, QROWS)], sem))
        for cp in cps:
            cp.wait()

        csv = cs_v[pl.ds(k * GE, GE)]
        cdv = cd_v[pl.ds(k * GE, GE)]
        csq = csv * csv
        cdq = cdv * cdv

        lane = lax.broadcasted_iota(jnp.int32, (16,), 0)
        scores = jnp.zeros((16,), _f32)
        for e in range(GE):
            def chunk(ch, accs, e=e):
                a0, a1, a2 = accs
                sl = pl.ds(ch * 16, 16)
                hs = rows_src[e, sl]
                hd = rows_dst[e, sl]
                S = rows_s[e * P, sl]
                Dv = rows_d[e * P, sl]
                for p in range(1, P):
                    S = S + rows_s[e * P + p, sl]
                    Dv = Dv + rows_d[e * P + p, sl]
                return (a0 + hs * hd, a1 + hd * S, a2 + hs * Dv)

            z = jnp.zeros((16,), _f32)
            a0, a1, a2 = lax.fori_loop(0, D // 16, chunk, (z, z, z))
            s_e = jnp.sum(a0) + csq[e] * jnp.sum(a1) + cdq[e] * jnp.sum(a2)
            scores = jnp.where(lane == e, s_e, scores)

        srcv = src_v[pl.ds(k * GE, GE)]
        dstv = dst_v[pl.ds(k * GE, GE)]
        nbs = plsc.load_gather(nb_v, [srcv + 1])
        nbd = plsc.load_gather(nb_v, [dstv + 1])
        out_v[pl.ds(k * GE, GE)] = scores + mu_vec + nbs + nbd
        return 0

    lax.fori_loop(0, NBLK, block, 0)
    pltpu.sync_copy(out_v, score_hbm.at[pl.ds(wid * BPW, BPW)])


_score_call = functools.partial(
    pl.kernel,
    out_type=jax.ShapeDtypeStruct((B,), _f32),
    mesh=plsc.VectorSubcoreMesh(core_axis_name="c", subcore_axis_name="s",
                                num_cores=NC, num_subcores=NS),
    compiler_params=pltpu.CompilerParams(needs_layout_passes=False),
    scratch_types=[
        pltpu.VMEM((BPW,), jnp.int32),
        pltpu.VMEM((BPW,), jnp.int32),
        pltpu.VMEM((BPW * P,), jnp.int32),
        pltpu.VMEM((BPW * P,), jnp.int32),
        pltpu.VMEM((BPW,), _f32),
        pltpu.VMEM((BPW,), _f32),
        pltpu.VMEM((NB_PAD,), _f32),
        pltpu.VMEM((16,), _f32),
        pltpu.VMEM((GE, D), _f32),
        pltpu.VMEM((GE, D), _f32),
        pltpu.VMEM((GE * P, D), _f32),
        pltpu.VMEM((GE * P, D), _f32),
        pltpu.VMEM((BPW,), _f32),
        pltpu.SemaphoreType.DMA,
    ],
)(_score_body)


def _kernel_debug_p1(x, edge_index, src, dst, s2d, s2dc, d2s, d2sc, W_self,
                     W_neigh, b, node_biases, mu):
    srcL = edge_index[0].reshape(NW, NCH, ECH)
    dstL = edge_index[1].reshape(NW, NCH, ECH)
    msum = _edge_call(x, srcL, dstL)
    dcnt = _deg_call(edge_index[1].reshape(NW, EPW))
    msgs = (msum[0] + msum[1])[:N_NODES]
    deg = dcnt.sum(axis=0)[:N_NODES]
    agg = msgs / jnp.clip(deg, 1.0)[:, None]
    h_output = jax.nn.relu(x @ W_self + agg @ W_neigh + b)
    h_src = h_output[src]
    h_dst = h_output[dst]
    s2d_imp = h_output[s2d] * (s2d != 0)[..., None].astype(_f32)
    d2s_imp = h_output[d2s] * (d2s != 0)[..., None].astype(_f32)
    s2d_term = s2dc * s2dc * (h_dst * s2d_imp.sum(axis=1)).sum(axis=1)
    d2s_term = d2sc * d2sc * (h_src * d2s_imp.sum(axis=1)).sum(axis=1)
    implicit = s2d_term + d2s_term
    return (mu + (h_src * h_dst).sum(axis=1) + node_biases[src + 1]
            + node_biases[dst + 1] + implicit)


def kernel(x, edge_index, src, dst, s2d, s2dc, d2s, d2sc, W_self, W_neigh, b,
           node_biases, mu):
    srcL = edge_index[0].reshape(NW, NCH, ECH)
    dstL = edge_index[1].reshape(NW, NCH, ECH)
    msum = _edge_call(x, srcL, dstL)
    dcnt = _deg_call(edge_index[1].reshape(NW, EPW))

    x_pad = jnp.pad(x, ((0, NPAD - N_NODES), (0, 0)))
    h = _h_call(x_pad, msum[0], msum[1], dcnt, W_self, W_neigh,
                b.reshape(1, D))

    srcr = src.reshape(NW, BPW)
    dstr = dst.reshape(NW, BPW)
    s2dr = s2d.reshape(NW, BPW * P)
    d2sr = d2s.reshape(NW, BPW * P)
    csr = s2dc.reshape(NW, BPW)
    cdr = d2sc.reshape(NW, BPW)
    nb = jnp.pad(node_biases, (0, NB_PAD - (N_NODES + 1)))
    mu_arr = jnp.broadcast_to(mu.astype(_f32), (16,))
    score = _score_call(h, srcr, dstr, s2dr, d2sr, csr, cdr, nb, mu_arr)
    return score
